# baseline (speedup 1.0000x reference)
import functools

import jax
import jax.numpy as jnp
import numpy as np
from jax import lax
from jax.experimental import pallas as pl
from jax.experimental.pallas import tpu as pltpu
from jax.experimental.pallas import tpu_sc as plsc

T = 8
M = 1.0
LAM = 0.01
BLK = 512
SC_CHUNK = 64


def _tc_a(v_ref, vh_ref, f_ref, neg_ref, mask_ref,
          h_ref, stats_ref, ortho_ref, csum_ref, msum_ref, acc):
    pid = pl.program_id(0)
    nblk = pl.num_programs(0)

    @pl.when(pid == 0)
    def _init():
        f = f_ref[...]
        gram = lax.dot_general(f, f, (((1,), (1,)), ((), ())),
                               preferred_element_type=jnp.float32)
        k = gram.shape[0]
        rows = lax.broadcasted_iota(jnp.int32, gram.shape, 0)
        cols = lax.broadcasted_iota(jnp.int32, gram.shape, 1)
        tr = jnp.sum(jnp.where(rows == cols, gram, 0.0))
        acc[0] = jnp.sum(gram * gram) - 2.0 * tr + float(k)
        acc[1] = 0.0
        acc[2] = 0.0

    vhat = vh_ref[...]
    v = v_ref[...]
    vh2 = jnp.sum(vhat * vhat, axis=1)
    td = jnp.sqrt(jnp.sum((vhat - v) ** 2, axis=1) + 1e-8)

    ones_row = jnp.ones((8, v.shape[1]), jnp.float32)
    neg = neg_ref[...]
    nn2 = lax.dot_general(ones_row, neg * neg, (((1,), (1,)), ((), ())),
                          preferred_element_type=jnp.float32)[0:1, :]
    ndots = lax.dot_general(vhat, neg, (((1,), (1,)), ((), ())),
                            preferred_element_type=jnp.float32)
    nd = jnp.sqrt(jnp.maximum(vh2[:, None] - 2.0 * ndots + nn2, 0.0) + 1e-8)
    c = jnp.mean(jnp.maximum(1.0 + td[:, None] - nd, 0.0), axis=1)

    f = f_ref[...]
    fn2 = lax.dot_general(ones_row, f * f, (((1,), (1,)), ((), ())),
                          preferred_element_type=jnp.float32)[0:1, :]
    dots = lax.dot_general(vhat, f, (((1,), (1,)), ((), ())),
                           preferred_element_type=jnp.float32)
    h_ref[...] = fn2 - 2.0 * dots

    zcol = jnp.zeros_like(td)
    stats_ref[...] = jnp.stack(
        [td, vh2, zcol, zcol, zcol, zcol, zcol, zcol], axis=1)

    mask = mask_ref[0, 0, :]
    acc[1] += jnp.sum(mask * c)
    acc[2] += jnp.sum(mask)

    @pl.when(pid == nblk - 1)
    def _fin():
        ortho_ref[...] = jnp.broadcast_to(acc[0], (1, 1))
        csum_ref[...] = jnp.broadcast_to(acc[1], (1, 1))
        msum_ref[...] = jnp.broadcast_to(acc[2], (1, 1))


def _sc_topk(g_hbm, gt_hbm, idx_hbm, g_v0, g_v1, gt_v, idx_v, sem0, sem1):
    g_bufs = [g_v0, g_v1]
    sem_bufs = [sem0, sem1]
    info = plsc.get_sparse_core_info()
    nc = info.num_cores
    nw = nc * info.num_subcores
    wid = lax.axis_index("s") * nc + lax.axis_index("c")
    rows_per_w = g_hbm.shape[0] // nw
    base = wid * rows_per_w
    kk = g_hbm.shape[1]
    nvec = kk // 16
    lane = lax.iota(jnp.int32, 16)

    def topk_row(gref, r):
        sorted_vecs = []
        for j in range(nvec):
            gv = gref[r, pl.ds(j * 16, 16)]
            kv = (plsc.bitcast(gv, jnp.int32) & np.int32(~511)) \
                | (lane + np.int32(j * 16))
            sorted_vecs.append(jnp.sort(kv))
        while len(sorted_vecs) > 1:
            nxt = []
            for a, b in zip(sorted_vecs[0::2], sorted_vecs[1::2]):
                low = jnp.minimum(a, jnp.flip(b, 0))
                nxt.append(jnp.sort(low))
            sorted_vecs = nxt
        return sorted_vecs[0]

    nchunk = rows_per_w // SC_CHUNK

    def issue(ci):
        buf = ci % 2
        rowbase = base + ci * SC_CHUNK
        return pltpu.async_copy(
            g_hbm.at[pl.ds(rowbase, SC_CHUNK)], g_bufs[buf], sem_bufs[buf])

    cp = issue(0)
    for ci in range(nchunk):
        buf = ci % 2
        nxt = issue(ci + 1) if ci + 1 < nchunk else None
        cp.wait()
        cp = nxt
        rowbase = base + ci * SC_CHUNK

        def do_row(r, carry, _buf=buf):
            for rr in (2 * r, 2 * r + 1):
                best = topk_row(g_bufs[_buf], rr)
                gval = plsc.bitcast(best & np.int32(~511), jnp.float32)
                off = pl.multiple_of(rr * 16, 16)
                gt_v[pl.ds(off, 16)] = gval
                idx_v[pl.ds(off, 16)] = best & np.int32(511)
            return carry

        lax.fori_loop(0, SC_CHUNK // 2, do_row, 0)
        pltpu.sync_copy(gt_v, gt_hbm.at[pl.ds(rowbase * 16, SC_CHUNK * 16)])
        pltpu.sync_copy(idx_v, idx_hbm.at[pl.ds(rowbase * 16, SC_CHUNK * 16)])


def _sc_jt(h_hbm, gt_hbm, idx_hbm, stats_hbm, mk_hbm, part_hbm,
           h_v0, h_v1, gt_v0, gt_v1, idx_v0, idx_v1, st_v0, st_v1,
           mk_v0, mk_v1, out_v, sem0, sem1):
    h_bufs = [h_v0, h_v1]
    gt_bufs = [gt_v0, gt_v1]
    idx_bufs = [idx_v0, idx_v1]
    st_bufs = [st_v0, st_v1]
    mk_bufs = [mk_v0, mk_v1]
    sem_bufs = [sem0, sem1]
    info = plsc.get_sparse_core_info()
    nc = info.num_cores
    nw = nc * info.num_subcores
    wid = lax.axis_index("s") * nc + lax.axis_index("c")
    rows_per_w = h_hbm.shape[0] // nw
    base = wid * rows_per_w
    lane = lax.iota(jnp.int32, 16)
    lt8 = lane < T
    zero16 = jnp.zeros((16,), jnp.int32)
    one16 = jnp.broadcast_to(jnp.int32(1), (16,))

    def do_row(href, gtref, idxref, stref, mkref, r, acc):
        off = pl.multiple_of(r * 16, 16)
        kidx = idxref[pl.ds(off, 16)]
        gval = gtref[pl.ds(off, 16)]
        rvec = jnp.broadcast_to(r, (16,)).astype(jnp.int32)
        hval = plsc.load_gather(href, [rvec, kidx])
        tdv = plsc.load_gather(stref, [rvec, zero16])
        vhv = plsc.load_gather(stref, [rvec, one16])
        mv = plsc.load_gather(mkref, [rvec])
        gm = jnp.where(lt8, gval, 0.0)
        s = jnp.broadcast_to(jnp.sum(gm), (16,))
        gn = gm / (s + 1e-10)
        one_m_gn = 1.0 - gn
        mt = M * one_m_gn * one_m_gn
        y = jnp.maximum(vhv + hval, 0.0) + 1e-8
        i0 = np.int32(0x5F3759DF) - lax.shift_right_arithmetic(
            plsc.bitcast(y, jnp.int32), 1)
        rs = plsc.bitcast(i0, jnp.float32)
        for _ in range(3):
            rs = rs * (1.5 - 0.5 * y * rs * rs)
        dist = y * rs
        term = jnp.maximum(mt + tdv - dist, 0.0) * (1.0 / T)
        return acc + jnp.where(lt8, term, 0.0) * mv

    nchunk = rows_per_w // SC_CHUNK

    def issue(ci):
        buf = ci % 2
        rowbase = base + ci * SC_CHUNK
        sem = sem_bufs[buf]
        return [
            pltpu.async_copy(h_hbm.at[pl.ds(rowbase, SC_CHUNK)],
                             h_bufs[buf], sem),
            pltpu.async_copy(gt_hbm.at[pl.ds(rowbase * 16, SC_CHUNK * 16)],
                             gt_bufs[buf], sem),
            pltpu.async_copy(idx_hbm.at[pl.ds(rowbase * 16, SC_CHUNK * 16)],
                             idx_bufs[buf], sem),
            pltpu.async_copy(stats_hbm.at[pl.ds(rowbase, SC_CHUNK)],
                             st_bufs[buf], sem),
            pltpu.async_copy(mk_hbm.at[pl.ds(rowbase, SC_CHUNK)],
                             mk_bufs[buf], sem),
        ]

    acc = jnp.zeros((16,), jnp.float32)
    cps = issue(0)
    for ci in range(nchunk):
        buf = ci % 2
        nxt = issue(ci + 1) if ci + 1 < nchunk else None
        for cp in cps:
            cp.wait()
        cps = nxt

        def row2(r, a, _buf=buf):
            a = do_row(h_bufs[_buf], gt_bufs[_buf], idx_bufs[_buf],
                       st_bufs[_buf], mk_bufs[_buf], 2 * r, a)
            return do_row(h_bufs[_buf], gt_bufs[_buf], idx_bufs[_buf],
                          st_bufs[_buf], mk_bufs[_buf], 2 * r + 1, a)

        acc = lax.fori_loop(0, SC_CHUNK // 2, row2, acc)
    out_v[...] = acc
    pltpu.sync_copy(out_v, part_hbm.at[pl.ds(wid * 16, 16)])


def _tc_b(part_ref, ortho_ref, csum_ref, msum_ref, out_ref):
    jts = jnp.sum(part_ref[...])
    val = (csum_ref[0, 0] + jts) / jnp.maximum(msum_ref[0, 0], 1.0) \
        + LAM * ortho_ref[0, 0]
    out_ref[...] = jnp.broadcast_to(val, (1, 1))


@functools.partial(jax.jit, static_argnames=())
def kernel(v, vhat, d, g, F, negatives, mask):
    del d
    B, D = v.shape
    K = F.shape[0]
    N = negatives.shape[0]
    nblk = B // BLK
    maskf = mask.astype(jnp.float32)

    mesh = plsc.VectorSubcoreMesh(core_axis_name="c", subcore_axis_name="s")
    nw = 32

    gt_flat, idx_flat = pl.kernel(
        _sc_topk,
        mesh=mesh,
        compiler_params=pltpu.CompilerParams(needs_layout_passes=False),
        out_type=[
            jax.ShapeDtypeStruct((B * 16,), jnp.float32),
            jax.ShapeDtypeStruct((B * 16,), jnp.int32),
        ],
        scratch_types=[
            pltpu.VMEM((SC_CHUNK, K), jnp.float32),
            pltpu.VMEM((SC_CHUNK, K), jnp.float32),
            pltpu.VMEM((SC_CHUNK * 16,), jnp.float32),
            pltpu.VMEM((SC_CHUNK * 16,), jnp.int32),
            pltpu.SemaphoreType.DMA,
            pltpu.SemaphoreType.DMA,
        ],
    )(g)

    h, stats, ortho, csum, msum = pl.pallas_call(
        _tc_a,
        grid=(nblk,),
        in_specs=[
            pl.BlockSpec((BLK, D), lambda i: (i, 0)),
            pl.BlockSpec((BLK, D), lambda i: (i, 0)),
            pl.BlockSpec((K, D), lambda i: (0, 0)),
            pl.BlockSpec((N, D), lambda i: (0, 0)),
            pl.BlockSpec((1, 1, BLK), lambda i: (i, 0, 0)),
        ],
        out_specs=[
            pl.BlockSpec((BLK, K), lambda i: (i, 0)),
            pl.BlockSpec((BLK, 8), lambda i: (i, 0)),
            pl.BlockSpec((1, 1), lambda i: (0, 0)),
            pl.BlockSpec((1, 1), lambda i: (0, 0)),
            pl.BlockSpec((1, 1), lambda i: (0, 0)),
        ],
        out_shape=[
            jax.ShapeDtypeStruct((B, K), jnp.float32),
            jax.ShapeDtypeStruct((B, 8), jnp.float32),
            jax.ShapeDtypeStruct((1, 1), jnp.float32),
            jax.ShapeDtypeStruct((1, 1), jnp.float32),
            jax.ShapeDtypeStruct((1, 1), jnp.float32),
        ],
        scratch_shapes=[pltpu.SMEM((3,), jnp.float32)],
    )(v, vhat, F, negatives, maskf.reshape(nblk, 1, BLK))

    partials = pl.kernel(
        _sc_jt,
        mesh=mesh,
        compiler_params=pltpu.CompilerParams(needs_layout_passes=False),
        out_type=jax.ShapeDtypeStruct((nw * 16,), jnp.float32),
        scratch_types=[
            pltpu.VMEM((SC_CHUNK, K), jnp.float32),
            pltpu.VMEM((SC_CHUNK, K), jnp.float32),
            pltpu.VMEM((SC_CHUNK * 16,), jnp.float32),
            pltpu.VMEM((SC_CHUNK * 16,), jnp.float32),
            pltpu.VMEM((SC_CHUNK * 16,), jnp.int32),
            pltpu.VMEM((SC_CHUNK * 16,), jnp.int32),
            pltpu.VMEM((SC_CHUNK, 8), jnp.float32),
            pltpu.VMEM((SC_CHUNK, 8), jnp.float32),
            pltpu.VMEM((SC_CHUNK,), jnp.float32),
            pltpu.VMEM((SC_CHUNK,), jnp.float32),
            pltpu.VMEM((16,), jnp.float32),
            pltpu.SemaphoreType.DMA,
            pltpu.SemaphoreType.DMA,
        ],
    )(h, gt_flat, idx_flat, stats, maskf)

    out = pl.pallas_call(
        _tc_b,
        in_specs=[
            pl.BlockSpec((nw * 16,), lambda: (0,)),
            pl.BlockSpec((1, 1), lambda: (0, 0)),
            pl.BlockSpec((1, 1), lambda: (0, 0)),
            pl.BlockSpec((1, 1), lambda: (0, 0)),
        ],
        out_specs=pl.BlockSpec((1, 1), lambda: (0, 0)),
        out_shape=jax.ShapeDtypeStruct((1, 1), jnp.float32),
    )(partials, ortho, csum, msum)
    return out.reshape(())


# speedup vs baseline: 1.0840x; 1.0840x over previous
import functools

import jax
import jax.numpy as jnp
import numpy as np
from jax import lax
from jax.experimental import pallas as pl
from jax.experimental.pallas import tpu as pltpu
from jax.experimental.pallas import tpu_sc as plsc

T = 8
M = 1.0
LAM = 0.01
BLK = 512
SC_CHUNK = 64
SC_ROWS = 14336


def _tc_a(v_ref, vh_ref, f_ref, neg_ref, mask_ref,
          h_ref, stats_ref, ortho_ref, csum_ref, msum_ref, acc):
    pid = pl.program_id(0)
    nblk = pl.num_programs(0)

    @pl.when(pid == 0)
    def _init():
        f = f_ref[...]
        gram = lax.dot_general(f, f, (((1,), (1,)), ((), ())),
                               preferred_element_type=jnp.float32)
        k = gram.shape[0]
        rows = lax.broadcasted_iota(jnp.int32, gram.shape, 0)
        cols = lax.broadcasted_iota(jnp.int32, gram.shape, 1)
        tr = jnp.sum(jnp.where(rows == cols, gram, 0.0))
        acc[0] = jnp.sum(gram * gram) - 2.0 * tr + float(k)
        acc[1] = 0.0
        acc[2] = 0.0

    vhat = vh_ref[...]
    v = v_ref[...]
    vh2 = jnp.sum(vhat * vhat, axis=1)
    td = jnp.sqrt(jnp.sum((vhat - v) ** 2, axis=1) + 1e-8)

    ones_row = jnp.ones((8, v.shape[1]), jnp.float32)
    neg = neg_ref[...]
    nn2 = lax.dot_general(ones_row, neg * neg, (((1,), (1,)), ((), ())),
                          preferred_element_type=jnp.float32)[0:1, :]
    ndots = lax.dot_general(vhat, neg, (((1,), (1,)), ((), ())),
                            preferred_element_type=jnp.float32)
    nd = jnp.sqrt(jnp.maximum(vh2[:, None] - 2.0 * ndots + nn2, 0.0) + 1e-8)
    c = jnp.mean(jnp.maximum(1.0 + td[:, None] - nd, 0.0), axis=1)

    f = f_ref[...]
    fn2 = lax.dot_general(ones_row, f * f, (((1,), (1,)), ((), ())),
                          preferred_element_type=jnp.float32)[0:1, :]
    dots = lax.dot_general(vhat, f, (((1,), (1,)), ((), ())),
                           preferred_element_type=jnp.float32)
    h_ref[...] = fn2 - 2.0 * dots

    zcol = jnp.zeros_like(td)
    stats_ref[...] = jnp.stack(
        [td, vh2, zcol, zcol, zcol, zcol, zcol, zcol], axis=1)

    mask = mask_ref[0, 0, :]
    acc[1] += jnp.sum(mask * c)
    acc[2] += jnp.sum(mask)

    @pl.when(pid == nblk - 1)
    def _fin():
        ortho_ref[...] = jnp.broadcast_to(acc[0], (1, 1))
        csum_ref[...] = jnp.broadcast_to(acc[1], (1, 1))
        msum_ref[...] = jnp.broadcast_to(acc[2], (1, 1))


def _sc_topk(g_hbm, gt_hbm, idx_hbm, g_v0, g_v1, gt_v, idx_v, sem0, sem1):
    g_bufs = [g_v0, g_v1]
    sem_bufs = [sem0, sem1]
    info = plsc.get_sparse_core_info()
    nc = info.num_cores
    nw = nc * info.num_subcores
    wid = lax.axis_index("s") * nc + lax.axis_index("c")
    rows_per_w = SC_ROWS // nw
    base = wid * rows_per_w
    kk = g_hbm.shape[1]
    nvec = kk // 16
    lane = lax.iota(jnp.int32, 16)

    def topk_row(gref, r):
        sorted_vecs = []
        for j in range(nvec):
            gv = gref[r, pl.ds(j * 16, 16)]
            kv = (plsc.bitcast(gv, jnp.int32) & np.int32(~511)) \
                | (lane + np.int32(j * 16))
            sorted_vecs.append(jnp.sort(kv))
        while len(sorted_vecs) > 1:
            nxt = []
            for a, b in zip(sorted_vecs[0::2], sorted_vecs[1::2]):
                low = jnp.minimum(a, jnp.flip(b, 0))
                nxt.append(jnp.sort(low))
            sorted_vecs = nxt
        return sorted_vecs[0]

    nchunk = rows_per_w // SC_CHUNK

    def issue(ci):
        buf = ci % 2
        rowbase = base + ci * SC_CHUNK
        return pltpu.async_copy(
            g_hbm.at[pl.ds(rowbase, SC_CHUNK)], g_bufs[buf], sem_bufs[buf])

    cp = issue(0)
    for ci in range(nchunk):
        buf = ci % 2
        nxt = issue(ci + 1) if ci + 1 < nchunk else None
        cp.wait()
        cp = nxt
        rowbase = base + ci * SC_CHUNK

        def do_row(r, carry, _buf=buf):
            for rr in (2 * r, 2 * r + 1):
                best = topk_row(g_bufs[_buf], rr)
                gval = plsc.bitcast(best & np.int32(~511), jnp.float32)
                off = pl.multiple_of(rr * 16, 16)
                gt_v[pl.ds(off, 16)] = gval
                idx_v[pl.ds(off, 16)] = best & np.int32(511)
            return carry

        lax.fori_loop(0, SC_CHUNK // 2, do_row, 0)
        pltpu.sync_copy(gt_v, gt_hbm.at[pl.ds(rowbase * 16, SC_CHUNK * 16)])
        pltpu.sync_copy(idx_v, idx_hbm.at[pl.ds(rowbase * 16, SC_CHUNK * 16)])


def _sc_jt(h_hbm, gt_hbm, idx_hbm, stats_hbm, mk_hbm, part_hbm,
           h_v0, h_v1, gt_v0, gt_v1, idx_v0, idx_v1, st_v0, st_v1,
           mk_v0, mk_v1, out_v, sem0, sem1):
    h_bufs = [h_v0, h_v1]
    gt_bufs = [gt_v0, gt_v1]
    idx_bufs = [idx_v0, idx_v1]
    st_bufs = [st_v0, st_v1]
    mk_bufs = [mk_v0, mk_v1]
    sem_bufs = [sem0, sem1]
    info = plsc.get_sparse_core_info()
    nc = info.num_cores
    nw = nc * info.num_subcores
    wid = lax.axis_index("s") * nc + lax.axis_index("c")
    rows_per_w = SC_ROWS // nw
    base = wid * rows_per_w
    lane = lax.iota(jnp.int32, 16)
    lt8 = lane < T
    zero16 = jnp.zeros((16,), jnp.int32)
    one16 = jnp.broadcast_to(jnp.int32(1), (16,))

    def do_row(href, gtref, idxref, stref, mkref, r, acc):
        off = pl.multiple_of(r * 16, 16)
        kidx = idxref[pl.ds(off, 16)]
        gval = gtref[pl.ds(off, 16)]
        rvec = jnp.broadcast_to(r, (16,)).astype(jnp.int32)
        hval = plsc.load_gather(href, [rvec, kidx])
        tdv = plsc.load_gather(stref, [rvec, zero16])
        vhv = plsc.load_gather(stref, [rvec, one16])
        mv = plsc.load_gather(mkref, [rvec])
        gm = jnp.where(lt8, gval, 0.0)
        s = jnp.broadcast_to(jnp.sum(gm), (16,))
        gn = gm / (s + 1e-10)
        one_m_gn = 1.0 - gn
        mt = M * one_m_gn * one_m_gn
        y = jnp.maximum(vhv + hval, 0.0) + 1e-8
        i0 = np.int32(0x5F3759DF) - lax.shift_right_arithmetic(
            plsc.bitcast(y, jnp.int32), 1)
        rs = plsc.bitcast(i0, jnp.float32)
        for _ in range(3):
            rs = rs * (1.5 - 0.5 * y * rs * rs)
        dist = y * rs
        term = jnp.maximum(mt + tdv - dist, 0.0) * (1.0 / T)
        return acc + jnp.where(lt8, term, 0.0) * mv

    nchunk = rows_per_w // SC_CHUNK

    def issue(ci):
        buf = ci % 2
        rowbase = base + ci * SC_CHUNK
        sem = sem_bufs[buf]
        return [
            pltpu.async_copy(h_hbm.at[pl.ds(rowbase, SC_CHUNK)],
                             h_bufs[buf], sem),
            pltpu.async_copy(gt_hbm.at[pl.ds(rowbase * 16, SC_CHUNK * 16)],
                             gt_bufs[buf], sem),
            pltpu.async_copy(idx_hbm.at[pl.ds(rowbase * 16, SC_CHUNK * 16)],
                             idx_bufs[buf], sem),
            pltpu.async_copy(stats_hbm.at[pl.ds(rowbase, SC_CHUNK)],
                             st_bufs[buf], sem),
            pltpu.async_copy(mk_hbm.at[pl.ds(rowbase, SC_CHUNK)],
                             mk_bufs[buf], sem),
        ]

    acc = jnp.zeros((16,), jnp.float32)
    cps = issue(0)
    for ci in range(nchunk):
        buf = ci % 2
        nxt = issue(ci + 1) if ci + 1 < nchunk else None
        for cp in cps:
            cp.wait()
        cps = nxt

        def row2(r, a, _buf=buf):
            a = do_row(h_bufs[_buf], gt_bufs[_buf], idx_bufs[_buf],
                       st_bufs[_buf], mk_bufs[_buf], 2 * r, a)
            return do_row(h_bufs[_buf], gt_bufs[_buf], idx_bufs[_buf],
                          st_bufs[_buf], mk_bufs[_buf], 2 * r + 1, a)

        acc = lax.fori_loop(0, SC_CHUNK // 2, row2, acc)
    out_v[...] = acc
    pltpu.sync_copy(out_v, part_hbm.at[pl.ds(wid * 16, 16)])


def _tc_c(v_ref, vh_ref, g_ref, f_ref, mask_ref, csum2_ref, acc):
    pid = pl.program_id(0)
    nblk = pl.num_programs(0)

    @pl.when(pid == 0)
    def _init():
        acc[0] = 0.0

    vhat = vh_ref[...]
    v = v_ref[...]
    vh2 = jnp.sum(vhat * vhat, axis=1)
    td = jnp.sqrt(jnp.sum((vhat - v) ** 2, axis=1) + 1e-8)
    ones_row = jnp.ones((8, v.shape[1]), jnp.float32)
    f = f_ref[...]
    fn2 = lax.dot_general(ones_row, f * f, (((1,), (1,)), ((), ())),
                          preferred_element_type=jnp.float32)[0:1, :]
    dots = lax.dot_general(vhat, f, (((1,), (1,)), ((), ())),
                           preferred_element_type=jnp.float32)
    h = fn2 - 2.0 * dots

    g = g_ref[...]
    gi = lax.bitcast_convert_type(g, jnp.int32)
    col = lax.broadcasted_iota(jnp.int32, g.shape, 1)
    keys = (gi & np.int32(~511)) | col
    gts, hts = [], []
    for _ in range(T):
        kmin = jnp.min(keys, axis=1)
        sel = keys == kmin[:, None]
        hts.append(jnp.sum(jnp.where(sel, h, 0.0), axis=1))
        gts.append(lax.bitcast_convert_type(kmin & np.int32(~511),
                                            jnp.float32))
        keys = jnp.where(sel, np.int32(2**31 - 1), keys)
    gt = jnp.stack(gts, axis=1)
    ht = jnp.stack(hts, axis=1)
    gn = gt / (jnp.sum(gt, axis=1, keepdims=True) + 1e-10)
    mt = M * (1.0 - gn) ** 2
    dist = jnp.sqrt(jnp.maximum(vh2[:, None] + ht, 0.0) + 1e-8)
    jt = jnp.mean(jnp.maximum(mt + td[:, None] - dist, 0.0), axis=1)

    mask = mask_ref[0, 0, :]
    acc[0] += jnp.sum(mask * jt)

    @pl.when(pid == nblk - 1)
    def _fin():
        csum2_ref[...] = jnp.broadcast_to(acc[0], (1, 1))


def _tc_b(part_ref, ortho_ref, csum_ref, csum2_ref, msum_ref, out_ref):
    jts = jnp.sum(part_ref[...])
    val = (csum_ref[0, 0] + csum2_ref[0, 0] + jts) \
        / jnp.maximum(msum_ref[0, 0], 1.0) + LAM * ortho_ref[0, 0]
    out_ref[...] = jnp.broadcast_to(val, (1, 1))


@functools.partial(jax.jit, static_argnames=())
def kernel(v, vhat, d, g, F, negatives, mask):
    del d
    B, D = v.shape
    K = F.shape[0]
    N = negatives.shape[0]
    nblk = B // BLK
    maskf = mask.astype(jnp.float32)

    mesh = plsc.VectorSubcoreMesh(core_axis_name="c", subcore_axis_name="s")
    nw = 32

    gt_flat, idx_flat = pl.kernel(
        _sc_topk,
        mesh=mesh,
        compiler_params=pltpu.CompilerParams(needs_layout_passes=False),
        out_type=[
            jax.ShapeDtypeStruct((SC_ROWS * 16,), jnp.float32),
            jax.ShapeDtypeStruct((SC_ROWS * 16,), jnp.int32),
        ],
        scratch_types=[
            pltpu.VMEM((SC_CHUNK, K), jnp.float32),
            pltpu.VMEM((SC_CHUNK, K), jnp.float32),
            pltpu.VMEM((SC_CHUNK * 16,), jnp.float32),
            pltpu.VMEM((SC_CHUNK * 16,), jnp.int32),
            pltpu.SemaphoreType.DMA,
            pltpu.SemaphoreType.DMA,
        ],
    )(g)

    h, stats, ortho, csum, msum = pl.pallas_call(
        _tc_a,
        grid=(nblk,),
        in_specs=[
            pl.BlockSpec((BLK, D), lambda i: (i, 0)),
            pl.BlockSpec((BLK, D), lambda i: (i, 0)),
            pl.BlockSpec((K, D), lambda i: (0, 0)),
            pl.BlockSpec((N, D), lambda i: (0, 0)),
            pl.BlockSpec((1, 1, BLK), lambda i: (i, 0, 0)),
        ],
        out_specs=[
            pl.BlockSpec((BLK, K), lambda i: (i, 0)),
            pl.BlockSpec((BLK, 8), lambda i: (i, 0)),
            pl.BlockSpec((1, 1), lambda i: (0, 0)),
            pl.BlockSpec((1, 1), lambda i: (0, 0)),
            pl.BlockSpec((1, 1), lambda i: (0, 0)),
        ],
        out_shape=[
            jax.ShapeDtypeStruct((B, K), jnp.float32),
            jax.ShapeDtypeStruct((B, 8), jnp.float32),
            jax.ShapeDtypeStruct((1, 1), jnp.float32),
            jax.ShapeDtypeStruct((1, 1), jnp.float32),
            jax.ShapeDtypeStruct((1, 1), jnp.float32),
        ],
        scratch_shapes=[pltpu.SMEM((3,), jnp.float32)],
    )(v, vhat, F, negatives, maskf.reshape(nblk, 1, BLK))

    partials = pl.kernel(
        _sc_jt,
        mesh=mesh,
        compiler_params=pltpu.CompilerParams(needs_layout_passes=False),
        out_type=jax.ShapeDtypeStruct((nw * 16,), jnp.float32),
        scratch_types=[
            pltpu.VMEM((SC_CHUNK, K), jnp.float32),
            pltpu.VMEM((SC_CHUNK, K), jnp.float32),
            pltpu.VMEM((SC_CHUNK * 16,), jnp.float32),
            pltpu.VMEM((SC_CHUNK * 16,), jnp.float32),
            pltpu.VMEM((SC_CHUNK * 16,), jnp.int32),
            pltpu.VMEM((SC_CHUNK * 16,), jnp.int32),
            pltpu.VMEM((SC_CHUNK, 8), jnp.float32),
            pltpu.VMEM((SC_CHUNK, 8), jnp.float32),
            pltpu.VMEM((SC_CHUNK,), jnp.float32),
            pltpu.VMEM((SC_CHUNK,), jnp.float32),
            pltpu.VMEM((16,), jnp.float32),
            pltpu.SemaphoreType.DMA,
            pltpu.SemaphoreType.DMA,
        ],
    )(h, gt_flat, idx_flat, stats, maskf)

    ntcc = (B - SC_ROWS) // BLK
    off = SC_ROWS // BLK
    csum2 = pl.pallas_call(
        _tc_c,
        grid=(ntcc,),
        in_specs=[
            pl.BlockSpec((BLK, D), lambda i: (i + off, 0)),
            pl.BlockSpec((BLK, D), lambda i: (i + off, 0)),
            pl.BlockSpec((BLK, K), lambda i: (i + off, 0)),
            pl.BlockSpec((K, D), lambda i: (0, 0)),
            pl.BlockSpec((1, 1, BLK), lambda i: (i + off, 0, 0)),
        ],
        out_specs=pl.BlockSpec((1, 1), lambda i: (0, 0)),
        out_shape=jax.ShapeDtypeStruct((1, 1), jnp.float32),
        scratch_shapes=[pltpu.SMEM((1,), jnp.float32)],
    )(v, vhat, g, F, maskf.reshape(nblk, 1, BLK))

    out = pl.pallas_call(
        _tc_b,
        in_specs=[
            pl.BlockSpec((nw * 16,), lambda: (0,)),
            pl.BlockSpec((1, 1), lambda: (0, 0)),
            pl.BlockSpec((1, 1), lambda: (0, 0)),
            pl.BlockSpec((1, 1), lambda: (0, 0)),
            pl.BlockSpec((1, 1), lambda: (0, 0)),
        ],
        out_specs=pl.BlockSpec((1, 1), lambda: (0, 0)),
        out_shape=jax.ShapeDtypeStruct((1, 1), jnp.float32),
    )(partials, ortho, csum, csum2, msum)
    return out.reshape(())


# speedup vs baseline: 1.0958x; 1.0109x over previous
import functools

import jax
import jax.numpy as jnp
import numpy as np
from jax import lax
from jax.experimental import pallas as pl
from jax.experimental.pallas import tpu as pltpu
from jax.experimental.pallas import tpu_sc as plsc

T = 8
M = 1.0
LAM = 0.01
BLK = 512
SC_CHUNK = 64
SC_ROWS = 14336


def _tc_a(v_ref, vh_ref, f_ref, neg_ref, mask_ref,
          h_ref, stats_ref, ortho_ref, csum_ref, msum_ref, acc):
    pid = pl.program_id(0)
    nblk = pl.num_programs(0)

    @pl.when(pid == 0)
    def _init():
        f = f_ref[...]
        gram = lax.dot_general(f, f, (((1,), (1,)), ((), ())),
                               preferred_element_type=jnp.float32)
        k = gram.shape[0]
        rows = lax.broadcasted_iota(jnp.int32, gram.shape, 0)
        cols = lax.broadcasted_iota(jnp.int32, gram.shape, 1)
        tr = jnp.sum(jnp.where(rows == cols, gram, 0.0))
        acc[0] = jnp.sum(gram * gram) - 2.0 * tr + float(k)
        acc[1] = 0.0
        acc[2] = 0.0

    vhat = vh_ref[...]
    v = v_ref[...]
    vh2 = jnp.sum(vhat * vhat, axis=1)
    td = jnp.sqrt(jnp.sum((vhat - v) ** 2, axis=1) + 1e-8)

    ones_row = jnp.ones((8, v.shape[1]), jnp.float32)
    neg = neg_ref[...]
    nn2 = lax.dot_general(ones_row, neg * neg, (((1,), (1,)), ((), ())),
                          preferred_element_type=jnp.float32)[0:1, :]
    ndots = lax.dot_general(vhat, neg, (((1,), (1,)), ((), ())),
                            preferred_element_type=jnp.float32)
    nd = jnp.sqrt(jnp.maximum(vh2[:, None] - 2.0 * ndots + nn2, 0.0) + 1e-8)
    c = jnp.mean(jnp.maximum(1.0 + td[:, None] - nd, 0.0), axis=1)

    f = f_ref[...]
    fn2 = lax.dot_general(ones_row, f * f, (((1,), (1,)), ((), ())),
                          preferred_element_type=jnp.float32)[0:1, :]
    dots = lax.dot_general(vhat, f, (((1,), (1,)), ((), ())),
                           preferred_element_type=jnp.float32)
    h_ref[...] = fn2 - 2.0 * dots

    zcol = jnp.zeros_like(td)
    stats_ref[...] = jnp.stack(
        [td, vh2, zcol, zcol, zcol, zcol, zcol, zcol], axis=1)

    mask = mask_ref[0, 0, :]
    acc[1] += jnp.sum(mask * c)
    acc[2] += jnp.sum(mask)

    @pl.when(pid == nblk - 1)
    def _fin():
        ortho_ref[...] = jnp.broadcast_to(acc[0], (1, 1))
        csum_ref[...] = jnp.broadcast_to(acc[1], (1, 1))
        msum_ref[...] = jnp.broadcast_to(acc[2], (1, 1))


def _sc_topk(g_hbm, gt_hbm, idx_hbm, g_v0, g_v1, gt_v0, gt_v1,
             idx_v0, idx_v1, sem0, sem1, osem0, osem1):
    g_bufs = [g_v0, g_v1]
    gt_bufs = [gt_v0, gt_v1]
    idx_bufs = [idx_v0, idx_v1]
    sem_bufs = [sem0, sem1]
    osem_bufs = [osem0, osem1]
    info = plsc.get_sparse_core_info()
    nc = info.num_cores
    nw = nc * info.num_subcores
    wid = lax.axis_index("s") * nc + lax.axis_index("c")
    rows_per_w = SC_ROWS // nw
    base = wid * rows_per_w
    kk = g_hbm.shape[1]
    nvec = kk // 16
    lane = lax.iota(jnp.int32, 16)

    def topk_row(gref, r):
        sorted_vecs = []
        for j in range(nvec):
            gv = gref[r, pl.ds(j * 16, 16)]
            kv = (plsc.bitcast(gv, jnp.int32) & np.int32(~511)) \
                | (lane + np.int32(j * 16))
            sorted_vecs.append(jnp.sort(kv))
        while len(sorted_vecs) > 1:
            nxt = []
            for a, b in zip(sorted_vecs[0::2], sorted_vecs[1::2]):
                low = jnp.minimum(a, jnp.flip(b, 0))
                nxt.append(jnp.sort(low))
            sorted_vecs = nxt
        return sorted_vecs[0]

    nchunk = rows_per_w // SC_CHUNK

    def issue(ci):
        buf = ci % 2
        rowbase = base + ci * SC_CHUNK
        return pltpu.async_copy(
            g_hbm.at[pl.ds(rowbase, SC_CHUNK)], g_bufs[buf], sem_bufs[buf])

    cp = issue(0)
    ocps = [None, None]
    for ci in range(nchunk):
        buf = ci % 2
        nxt = issue(ci + 1) if ci + 1 < nchunk else None
        cp.wait()
        cp = nxt
        rowbase = base + ci * SC_CHUNK
        if ocps[buf] is not None:
            for ocp in ocps[buf]:
                ocp.wait()
            ocps[buf] = None

        def do_row(r, carry, _buf=buf):
            for rr in (2 * r, 2 * r + 1):
                best = topk_row(g_bufs[_buf], rr)
                gval = plsc.bitcast(best & np.int32(~511), jnp.float32)
                off = pl.multiple_of(rr * 16, 16)
                gt_bufs[_buf][pl.ds(off, 16)] = gval
                idx_bufs[_buf][pl.ds(off, 16)] = best & np.int32(511)
            return carry

        lax.fori_loop(0, SC_CHUNK // 2, do_row, 0)
        ocps[buf] = [
            pltpu.async_copy(
                gt_bufs[buf],
                gt_hbm.at[pl.ds(rowbase * 16, SC_CHUNK * 16)],
                osem_bufs[buf]),
            pltpu.async_copy(
                idx_bufs[buf],
                idx_hbm.at[pl.ds(rowbase * 16, SC_CHUNK * 16)],
                osem_bufs[buf]),
        ]
    for pair in ocps:
        if pair is not None:
            for ocp in pair:
                ocp.wait()


def _sc_jt(h_hbm, gt_hbm, idx_hbm, stats_hbm, mk_hbm, part_hbm,
           h_v0, h_v1, gt_v0, gt_v1, idx_v0, idx_v1, st_v0, st_v1,
           mk_v0, mk_v1, out_v, sem0, sem1):
    h_bufs = [h_v0, h_v1]
    gt_bufs = [gt_v0, gt_v1]
    idx_bufs = [idx_v0, idx_v1]
    st_bufs = [st_v0, st_v1]
    mk_bufs = [mk_v0, mk_v1]
    sem_bufs = [sem0, sem1]
    info = plsc.get_sparse_core_info()
    nc = info.num_cores
    nw = nc * info.num_subcores
    wid = lax.axis_index("s") * nc + lax.axis_index("c")
    rows_per_w = SC_ROWS // nw
    base = wid * rows_per_w
    lane = lax.iota(jnp.int32, 16)
    lt8 = lane < T
    zero16 = jnp.zeros((16,), jnp.int32)
    one16 = jnp.broadcast_to(jnp.int32(1), (16,))

    def do_row(href, gtref, idxref, stref, mkref, r, acc):
        off = pl.multiple_of(r * 16, 16)
        kidx = idxref[pl.ds(off, 16)]
        gval = gtref[pl.ds(off, 16)]
        rvec = jnp.broadcast_to(r, (16,)).astype(jnp.int32)
        hval = plsc.load_gather(href, [rvec, kidx])
        tdv = plsc.load_gather(stref, [rvec, zero16])
        vhv = plsc.load_gather(stref, [rvec, one16])
        mv = plsc.load_gather(mkref, [rvec])
        gm = jnp.where(lt8, gval, 0.0)
        s = jnp.broadcast_to(jnp.sum(gm), (16,))
        gn = gm / (s + 1e-10)
        one_m_gn = 1.0 - gn
        mt = M * one_m_gn * one_m_gn
        y = jnp.maximum(vhv + hval, 0.0) + 1e-8
        i0 = np.int32(0x5F3759DF) - lax.shift_right_arithmetic(
            plsc.bitcast(y, jnp.int32), 1)
        rs = plsc.bitcast(i0, jnp.float32)
        for _ in range(3):
            rs = rs * (1.5 - 0.5 * y * rs * rs)
        dist = y * rs
        term = jnp.maximum(mt + tdv - dist, 0.0) * (1.0 / T)
        return acc + jnp.where(lt8, term, 0.0) * mv

    nchunk = rows_per_w // SC_CHUNK

    def issue(ci):
        buf = ci % 2
        rowbase = base + ci * SC_CHUNK
        sem = sem_bufs[buf]
        return [
            pltpu.async_copy(h_hbm.at[pl.ds(rowbase, SC_CHUNK)],
                             h_bufs[buf], sem),
            pltpu.async_copy(gt_hbm.at[pl.ds(rowbase * 16, SC_CHUNK * 16)],
                             gt_bufs[buf], sem),
            pltpu.async_copy(idx_hbm.at[pl.ds(rowbase * 16, SC_CHUNK * 16)],
                             idx_bufs[buf], sem),
            pltpu.async_copy(stats_hbm.at[pl.ds(rowbase, SC_CHUNK)],
                             st_bufs[buf], sem),
            pltpu.async_copy(mk_hbm.at[pl.ds(rowbase, SC_CHUNK)],
                             mk_bufs[buf], sem),
        ]

    acc = jnp.zeros((16,), jnp.float32)
    cps = issue(0)
    for ci in range(nchunk):
        buf = ci % 2
        nxt = issue(ci + 1) if ci + 1 < nchunk else None
        for cp in cps:
            cp.wait()
        cps = nxt

        def row2(r, a, _buf=buf):
            a = do_row(h_bufs[_buf], gt_bufs[_buf], idx_bufs[_buf],
                       st_bufs[_buf], mk_bufs[_buf], 2 * r, a)
            return do_row(h_bufs[_buf], gt_bufs[_buf], idx_bufs[_buf],
                          st_bufs[_buf], mk_bufs[_buf], 2 * r + 1, a)

        acc = lax.fori_loop(0, SC_CHUNK // 2, row2, acc)
    out_v[...] = acc
    pltpu.sync_copy(out_v, part_hbm.at[pl.ds(wid * 16, 16)])


def _tc_c(v_ref, vh_ref, g_ref, f_ref, mask_ref, csum2_ref, acc):
    pid = pl.program_id(0)
    nblk = pl.num_programs(0)

    @pl.when(pid == 0)
    def _init():
        acc[0] = 0.0

    vhat = vh_ref[...]
    v = v_ref[...]
    vh2 = jnp.sum(vhat * vhat, axis=1)
    td = jnp.sqrt(jnp.sum((vhat - v) ** 2, axis=1) + 1e-8)
    ones_row = jnp.ones((8, v.shape[1]), jnp.float32)
    f = f_ref[...]
    fn2 = lax.dot_general(ones_row, f * f, (((1,), (1,)), ((), ())),
                          preferred_element_type=jnp.float32)[0:1, :]
    dots = lax.dot_general(vhat, f, (((1,), (1,)), ((), ())),
                           preferred_element_type=jnp.float32)
    h = fn2 - 2.0 * dots

    g = g_ref[...]
    gi = lax.bitcast_convert_type(g, jnp.int32)
    col = lax.broadcasted_iota(jnp.int32, g.shape, 1)
    keys = (gi & np.int32(~511)) | col
    gts, hts = [], []
    for _ in range(T):
        kmin = jnp.min(keys, axis=1)
        sel = keys == kmin[:, None]
        hts.append(jnp.sum(jnp.where(sel, h, 0.0), axis=1))
        gts.append(lax.bitcast_convert_type(kmin & np.int32(~511),
                                            jnp.float32))
        keys = jnp.where(sel, np.int32(2**31 - 1), keys)
    gt = jnp.stack(gts, axis=1)
    ht = jnp.stack(hts, axis=1)
    gn = gt / (jnp.sum(gt, axis=1, keepdims=True) + 1e-10)
    mt = M * (1.0 - gn) ** 2
    dist = jnp.sqrt(jnp.maximum(vh2[:, None] + ht, 0.0) + 1e-8)
    jt = jnp.mean(jnp.maximum(mt + td[:, None] - dist, 0.0), axis=1)

    mask = mask_ref[0, 0, :]
    acc[0] += jnp.sum(mask * jt)

    @pl.when(pid == nblk - 1)
    def _fin():
        csum2_ref[...] = jnp.broadcast_to(acc[0], (1, 1))


def _tc_b(part_ref, ortho_ref, csum_ref, csum2_ref, msum_ref, out_ref):
    jts = jnp.sum(part_ref[...])
    val = (csum_ref[0, 0] + csum2_ref[0, 0] + jts) \
        / jnp.maximum(msum_ref[0, 0], 1.0) + LAM * ortho_ref[0, 0]
    out_ref[...] = jnp.broadcast_to(val, (1, 1))


@functools.partial(jax.jit, static_argnames=())
def kernel(v, vhat, d, g, F, negatives, mask):
    del d
    B, D = v.shape
    K = F.shape[0]
    N = negatives.shape[0]
    nblk = B // BLK
    maskf = mask.astype(jnp.float32)

    mesh = plsc.VectorSubcoreMesh(core_axis_name="c", subcore_axis_name="s")
    nw = 32

    gt_flat, idx_flat = pl.kernel(
        _sc_topk,
        mesh=mesh,
        compiler_params=pltpu.CompilerParams(needs_layout_passes=False),
        out_type=[
            jax.ShapeDtypeStruct((SC_ROWS * 16,), jnp.float32),
            jax.ShapeDtypeStruct((SC_ROWS * 16,), jnp.int32),
        ],
        scratch_types=[
            pltpu.VMEM((SC_CHUNK, K), jnp.float32),
            pltpu.VMEM((SC_CHUNK, K), jnp.float32),
            pltpu.VMEM((SC_CHUNK * 16,), jnp.float32),
            pltpu.VMEM((SC_CHUNK * 16,), jnp.float32),
            pltpu.VMEM((SC_CHUNK * 16,), jnp.int32),
            pltpu.VMEM((SC_CHUNK * 16,), jnp.int32),
            pltpu.SemaphoreType.DMA,
            pltpu.SemaphoreType.DMA,
            pltpu.SemaphoreType.DMA,
            pltpu.SemaphoreType.DMA,
        ],
    )(g)

    h, stats, ortho, csum, msum = pl.pallas_call(
        _tc_a,
        grid=(nblk,),
        in_specs=[
            pl.BlockSpec((BLK, D), lambda i: (i, 0)),
            pl.BlockSpec((BLK, D), lambda i: (i, 0)),
            pl.BlockSpec((K, D), lambda i: (0, 0)),
            pl.BlockSpec((N, D), lambda i: (0, 0)),
            pl.BlockSpec((1, 1, BLK), lambda i: (i, 0, 0)),
        ],
        out_specs=[
            pl.BlockSpec((BLK, K), lambda i: (i, 0)),
            pl.BlockSpec((BLK, 8), lambda i: (i, 0)),
            pl.BlockSpec((1, 1), lambda i: (0, 0)),
            pl.BlockSpec((1, 1), lambda i: (0, 0)),
            pl.BlockSpec((1, 1), lambda i: (0, 0)),
        ],
        out_shape=[
            jax.ShapeDtypeStruct((B, K), jnp.float32),
            jax.ShapeDtypeStruct((B, 8), jnp.float32),
            jax.ShapeDtypeStruct((1, 1), jnp.float32),
            jax.ShapeDtypeStruct((1, 1), jnp.float32),
            jax.ShapeDtypeStruct((1, 1), jnp.float32),
        ],
        scratch_shapes=[pltpu.SMEM((3,), jnp.float32)],
    )(v, vhat, F, negatives, maskf.reshape(nblk, 1, BLK))

    partials = pl.kernel(
        _sc_jt,
        mesh=mesh,
        compiler_params=pltpu.CompilerParams(needs_layout_passes=False),
        out_type=jax.ShapeDtypeStruct((nw * 16,), jnp.float32),
        scratch_types=[
            pltpu.VMEM((SC_CHUNK, K), jnp.float32),
            pltpu.VMEM((SC_CHUNK, K), jnp.float32),
            pltpu.VMEM((SC_CHUNK * 16,), jnp.float32),
            pltpu.VMEM((SC_CHUNK * 16,), jnp.float32),
            pltpu.VMEM((SC_CHUNK * 16,), jnp.int32),
            pltpu.VMEM((SC_CHUNK * 16,), jnp.int32),
            pltpu.VMEM((SC_CHUNK, 8), jnp.float32),
            pltpu.VMEM((SC_CHUNK, 8), jnp.float32),
            pltpu.VMEM((SC_CHUNK,), jnp.float32),
            pltpu.VMEM((SC_CHUNK,), jnp.float32),
            pltpu.VMEM((16,), jnp.float32),
            pltpu.SemaphoreType.DMA,
            pltpu.SemaphoreType.DMA,
        ],
    )(h, gt_flat, idx_flat, stats, maskf)

    ntcc = (B - SC_ROWS) // BLK
    off = SC_ROWS // BLK
    csum2 = pl.pallas_call(
        _tc_c,
        grid=(ntcc,),
        in_specs=[
            pl.BlockSpec((BLK, D), lambda i: (i + off, 0)),
            pl.BlockSpec((BLK, D), lambda i: (i + off, 0)),
            pl.BlockSpec((BLK, K), lambda i: (i + off, 0)),
            pl.BlockSpec((K, D), lambda i: (0, 0)),
            pl.BlockSpec((1, 1, BLK), lambda i: (i + off, 0, 0)),
        ],
        out_specs=pl.BlockSpec((1, 1), lambda i: (0, 0)),
        out_shape=jax.ShapeDtypeStruct((1, 1), jnp.float32),
        scratch_shapes=[pltpu.SMEM((1,), jnp.float32)],
    )(v, vhat, g, F, maskf.reshape(nblk, 1, BLK))

    out = pl.pallas_call(
        _tc_b,
        in_specs=[
            pl.BlockSpec((nw * 16,), lambda: (0,)),
            pl.BlockSpec((1, 1), lambda: (0, 0)),
            pl.BlockSpec((1, 1), lambda: (0, 0)),
            pl.BlockSpec((1, 1), lambda: (0, 0)),
            pl.BlockSpec((1, 1), lambda: (0, 0)),
        ],
        out_specs=pl.BlockSpec((1, 1), lambda: (0, 0)),
        out_shape=jax.ShapeDtypeStruct((1, 1), jnp.float32),
    )(partials, ortho, csum, csum2, msum)
    return out.reshape(())
